# Initial kernel scaffold; baseline (speedup 1.0000x reference)
import jax
import jax.numpy as jnp
from jax.experimental import pallas as pl


def kernel(features):
    raise NotImplementedError("write your pallas kernel here")


# speedup vs baseline: 1.0169x; 1.0169x over previous
import jax
import jax.numpy as jnp
from jax.experimental import pallas as pl

NT = 64


def _pool_body(x_ref, o_ref):
    win = x_ref.shape[2]
    o_ref[0, :, :] = jnp.sum(x_ref[0], axis=1) * (1.0 / win)


def kernel(features):
    B, N, D = features.shape
    win = N // NT
    x = features.reshape(B, NT, win, D)
    out = pl.pallas_call(
        _pool_body,
        grid=(B,),
        in_specs=[pl.BlockSpec((1, NT, win, D), lambda b: (b, 0, 0, 0))],
        out_specs=pl.BlockSpec((1, NT, D), lambda b: (b, 0, 0)),
        out_shape=jax.ShapeDtypeStruct((B, NT, D), jnp.float32),
    )(x)
    return out
